# baseline (speedup 1.0000x reference)
import functools

import jax
import jax.numpy as jnp
from jax import lax
from jax.experimental import pallas as pl
from jax.experimental.pallas import tpu as pltpu
from jax.experimental.pallas import tpu_sc as plsc

NC = 2
NS = 16
NW = NC * NS
RB = 8000


def _tc_row_sums(emb3):
    Q, _, D = emb3.shape
    QB = RB // 64
    assert D == 64 and Q % QB == 0

    def body(x_ref, o_ref):
        x = x_ref[...].reshape(QB * 64, D).astype(jnp.bfloat16)
        ones = jnp.ones((D, 64), jnp.bfloat16)
        o1 = jnp.dot(x, ones, preferred_element_type=jnp.float32)
        o_ref[...] = o1[:, :1].reshape(1, QB, 64)

    return pl.pallas_call(
        body,
        grid=(Q // QB,),
        in_specs=[pl.BlockSpec((QB, 64, D), lambda i: (i, 0, 0))],
        out_specs=pl.BlockSpec((1, QB, 64), lambda i: (i, 0, 0)),
        out_shape=jax.ShapeDtypeStruct((Q // QB, QB, 64), jnp.float32),
    )(emb3)


def _sc_gather(pos_words, neg_words, rs_flat):
    B = pos_words.shape[0]
    bpw = B // NW
    nch = bpw // 128

    mesh = plsc.VectorSubcoreMesh(core_axis_name="c", subcore_axis_name="s")

    @functools.partial(
        pl.kernel,
        mesh=mesh,
        compiler_params=pltpu.CompilerParams(needs_layout_passes=False,
                                             use_tc_tiling_on_sc=False),
        out_type=[
            jax.ShapeDtypeStruct((B,), jnp.float32),
            jax.ShapeDtypeStruct((B,), jnp.float32),
        ],
        scratch_types=[
            pltpu.VMEM((nch, 128), jnp.int32),
            pltpu.VMEM((nch, 128), jnp.int32),
            pltpu.VMEM((nch, 128), jnp.float32),
            pltpu.VMEM((nch, 128), jnp.float32),
            pltpu.SemaphoreType.DMA,
        ],
    )
    def sc_kernel(pos_hbm, neg_hbm, rs_hbm, pos_out, neg_out,
                  pidx, nidx, pval, nval, sem):
        wid = lax.axis_index("s") * NC + lax.axis_index("c")
        base = wid * bpw

        for j in range(nch):
            pltpu.sync_copy(pos_hbm.at[pl.ds(base + j * 128, 128)],
                            pidx.at[j])
            pltpu.sync_copy(neg_hbm.at[pl.ds(base + j * 128, 128)],
                            nidx.at[j])
        copies = [
            pltpu.make_async_copy(rs_hbm.at[pidx.at[j]], pval.at[j], sem)
            for j in range(nch)
        ] + [
            pltpu.make_async_copy(rs_hbm.at[nidx.at[j]], nval.at[j], sem)
            for j in range(nch)
        ]
        for c in copies:
            c.start()
        for c in copies:
            c.wait()
        for j in range(nch):
            pltpu.sync_copy(pval.at[j],
                            pos_out.at[pl.ds(base + j * 128, 128)])
            pltpu.sync_copy(nval.at[j],
                            neg_out.at[pl.ds(base + j * 128, 128)])

    return sc_kernel(pos_words, neg_words, rs_flat)


def _finish(pos_sums, neg_sums, batch):
    def body(p_ref, n_ref, o_ref):
        p = p_ref[...]
        n = n_ref[...]
        t = jnp.maximum(-p, 0.0) + jnp.log(1.0 + jnp.exp(-jnp.abs(p)))
        t = t + jnp.maximum(n, 0.0) + jnp.log(1.0 + jnp.exp(-jnp.abs(n)))
        o_ref[0, 0] = jnp.sum(t) * (1.0 / batch)

    out = pl.pallas_call(
        body,
        out_shape=jax.ShapeDtypeStruct((1, 1), jnp.float32),
        out_specs=pl.BlockSpec(memory_space=pltpu.SMEM),
    )(pos_sums, neg_sums)
    return out[0, 0]


def kernel(pos_words, neg_words, embeddings):
    B = pos_words.shape[0]
    V, D = embeddings.shape
    rs = _tc_row_sums(embeddings.reshape(V // 64, 64, D))
    rs_flat = rs.reshape(V)
    pos_sums, neg_sums = _sc_gather(pos_words.astype(jnp.int32),
                                    neg_words.astype(jnp.int32),
                                    rs_flat)
    return _finish(pos_sums.reshape(128, -1), neg_sums.reshape(128, -1), B)
